# baseline (speedup 1.0000x reference)
import functools

import numpy as np
import jax
import jax.numpy as jnp
from jax import lax
from jax.experimental import pallas as pl
from jax.experimental.pallas import tpu as pltpu
from jax.experimental.pallas import tpu_sc as plsc

N = 10000
E = 320000
D = 128
F = 64

NC, NS = 2, 16
NW = NC * NS
GB = 128
EP = 327680
EPWP = EP // NW
NBP = EPWP // GB
CB = 8
NCH = NBP // CB
ACC_ROWS = N + 128

ROWS_PER_TILE = 624

BE = 2560
BN = 2000


def _xm_body(x_ref, wm_ref, bm_ref, o_ref):
    o_ref[:, :] = (
        jnp.dot(x_ref[:, :], wm_ref[:, :], preferred_element_type=jnp.float32)
        + bm_ref[:, :]
    )


_CPOLY = (0.9999710833, -19.73279612, 64.7143697, -82.70120228,
          46.31009229)
_SPOLY = (6.283168274, -41.33792977, 81.47313282, -75.09327405,
          33.95650071)
_NPC = len(_CPOLY)
_LC = np.stack(
    [np.concatenate([np.full(F, c, np.float32), np.full(F, s, np.float32)])
     for c, s in zip(_CPOLY, _SPOLY)]
    + [np.concatenate([np.ones(F, np.float32), np.zeros(F, np.float32)]),
       np.concatenate([np.zeros(F, np.float32), np.ones(F, np.float32)]),
       np.zeros(2 * F, np.float32)])


def _fourier_body(rt_ref, frt_ref, lct_ref, wcst_ref, wlastt_ref, bfet_ref,
                  wf2_ref, bf2_ref, o_ref):
    ht = jnp.zeros((D, BE), dtype=jnp.float32)
    for i in range(3):
        ri = rt_ref[i:i + 1, :]
        t = frt_ref[:, i:i + 1] * ri
        u = t - jnp.round(t)
        v = u * u
        p = lct_ref[:, _NPC - 1:_NPC] + jnp.zeros_like(v)
        for k in range(_NPC - 2, -1, -1):
            p = p * v + lct_ref[:, k:k + 1]
        featt = p * (lct_ref[:, _NPC:_NPC + 1]
                     + lct_ref[:, _NPC + 1:_NPC + 2] * u)
        acc = jnp.dot(wcst_ref[i], featt, preferred_element_type=jnp.float32)
        acc = acc + wlastt_ref[:, i:i + 1] * ri + bfet_ref[:, i:i + 1]
        ht = ht + jnp.maximum(acc, 0.0)
    emb = lax.dot_general(ht, wf2_ref[:, :], (((0,), (0,)), ((), ())),
                          preferred_element_type=jnp.float32)
    o_ref[:, :] = jnp.maximum(emb + bf2_ref[:, :], 0.0)


def _tail_body(x_ref, a_ref, wo_ref, lng_ref, lnb_ref, wh1_ref, bh1_ref,
               wh2_ref, bh2_ref, o_ref):
    agg = a_ref[0] + a_ref[1]
    h = x_ref[:, :] + jnp.dot(agg, wo_ref[:, :],
                              preferred_element_type=jnp.float32)
    mu = jnp.mean(h, axis=1, keepdims=True)
    var = jnp.mean((h - mu) ** 2, axis=1, keepdims=True)
    hn = (h - mu) * lax.rsqrt(var + 1e-5) * lng_ref[:, :] + lnb_ref[:, :]
    t = jnp.maximum(
        jnp.dot(hn, wh1_ref[:, :], preferred_element_type=jnp.float32)
        + bh1_ref[:, :], 0.0)
    o_ref[:, :] = (
        jnp.dot(t, wh2_ref[:, :], preferred_element_type=jnp.float32)
        + bh2_ref[:, :]
    )


def _sc_body(xm_hbm, pre_hbm, src_hbm, dst_hbm, out_hbm,
             src_v, dst_v, gbuf, pbuf, zbuf, acc, gsem, ssem):
    c = lax.axis_index("c")
    s = lax.axis_index("s")
    w = s * NC + c

    zv = jnp.zeros((16,), jnp.float32)
    for i in range(8):
        for g in range(D // 16):
            zbuf[i, pl.ds(g * 16, 16)] = zv

    row0 = s * ROWS_PER_TILE
    nzc = jnp.where(s == NS - 1, (ACC_ROWS - 15 * ROWS_PER_TILE) // 8,
                    ROWS_PER_TILE // 8)

    def zacc(k, carry):
        pltpu.sync_copy(zbuf, acc.at[pl.ds(row0 + k * 8, 8)])
        return carry

    lax.fori_loop(0, nzc, zacc, 0)
    plsc.subcore_barrier()

    def chunk(cc, carry):
        base = (cc * NW + w) * CB
        pltpu.sync_copy(src_hbm.at[pl.ds(base, CB)], src_v)
        pltpu.sync_copy(dst_hbm.at[pl.ds(base, CB)], dst_v)

        g = pltpu.async_copy(xm_hbm.at[src_v.at[0]], gbuf, gsem)
        s_prev = None
        for j in range(CB):
            if s_prev is not None:
                s_prev.wait()
            prow = jnp.minimum(base + j, E // GB - 1) * GB
            pltpu.sync_copy(pre_hbm.at[pl.ds(prow, GB)], pbuf)
            g.wait()

            @plsc.parallel_loop(0, GB, unroll=4)
            def rowfn(i):
                for gg in range(D // 16):
                    sl = pl.ds(gg * 16, 16)
                    pbuf[i, sl] = jnp.maximum(gbuf[i, sl] + pbuf[i, sl], 0.0)

            if j < CB - 1:
                g = pltpu.async_copy(xm_hbm.at[src_v.at[j + 1]], gbuf, gsem)
            s_prev = pltpu.async_copy(pbuf, acc.at[dst_v.at[j]], ssem,
                                      add=True)
        s_prev.wait()
        return carry

    lax.fori_loop(0, NCH, chunk, 0)
    plsc.subcore_barrier()

    @pl.when(s == 0)
    def _flush():
        pltpu.sync_copy(acc.at[pl.ds(0, N)], out_hbm.at[c])


@functools.cache
def _get_sc_call():
    return functools.partial(
        pl.kernel,
    out_type=jax.ShapeDtypeStruct((NC, N, D), jnp.float32),
        mesh=plsc.VectorSubcoreMesh(core_axis_name="c", subcore_axis_name="s",
                                    num_cores=NC, num_subcores=NS),
        scratch_types=[
            pltpu.VMEM((CB, GB), jnp.int32),
            pltpu.VMEM((CB, GB), jnp.int32),
            pltpu.VMEM((GB, D), jnp.float32),
            pltpu.VMEM((GB, D), jnp.float32),
            pltpu.VMEM((8, D), jnp.float32),
            pltpu.VMEM_SHARED((ACC_ROWS, D), jnp.float32),
            pltpu.SemaphoreType.DMA,
            pltpu.SemaphoreType.DMA,
        ],
    )(_sc_body)


def kernel(x, edge_index, r, freqs, W_fe, b_fe, W_f2, b_f2, Wm, bm, Wo,
           ln_g, ln_b, W_h1, b_h1, W_h2, b_h2):
    pad_src = jnp.arange(EP - E, dtype=jnp.int32) % N
    srcp = jnp.concatenate([edge_index[0], pad_src]).reshape(NW * NBP, GB)
    pad_dst = N + (jnp.arange(EP - E, dtype=jnp.int32) % 128)
    dstp = jnp.concatenate([edge_index[1], pad_dst]).reshape(NW * NBP, GB)
    Wcs = W_fe[:, :2 * F, :]
    Wlast = W_fe[:, 2 * F, :]
    bm2 = bm.reshape(1, D)
    bf2 = b_f2.reshape(1, D)
    lng2 = ln_g.reshape(1, D)
    lnb2 = ln_b.reshape(1, D)
    W_h1p = jnp.pad(W_h1, ((0, 0), (0, 28)))
    b_h1p = jnp.pad(b_h1, (0, 28)).reshape(1, 128)
    W_h2p = jnp.pad(W_h2, ((0, 28), (0, 84)))
    b_h2p = jnp.pad(b_h2, (0, 84)).reshape(1, 384)

    full = lambda shape: pl.BlockSpec(shape, lambda i: tuple(0 for _ in shape))

    xm = pl.pallas_call(
        _xm_body,
        grid=(N // BN,),
        in_specs=[
            pl.BlockSpec((BN, D), lambda i: (i, 0)),
            full((D, D)),
            full((1, D)),
        ],
        out_specs=pl.BlockSpec((BN, D), lambda i: (i, 0)),
        out_shape=jax.ShapeDtypeStruct((N, D), jnp.float32),
    )(x, Wm, bm2)

    rt = jnp.swapaxes(r, 0, 1)
    frt = jnp.swapaxes(jnp.concatenate([freqs, freqs], axis=1), 0, 1)
    pre = pl.pallas_call(
        _fourier_body,
        grid=(E // BE,),
        in_specs=[
            pl.BlockSpec((3, BE), lambda i: (0, i)),
            full((D, 3)),
            full((D, 8)),
            full((3, D, 2 * F)),
            full((D, 3)),
            full((D, 3)),
            full((D, D)),
            full((1, D)),
        ],
        out_specs=pl.BlockSpec((BE, D), lambda i: (i, 0)),
        out_shape=jax.ShapeDtypeStruct((E, D), jnp.float32),
    )(rt, frt, jnp.asarray(_LC.T.copy()), jnp.swapaxes(Wcs, 1, 2),
      Wlast.T, b_fe.T, W_f2, bf2)

    parts = _get_sc_call()(xm, pre, srcp, dstp)

    occ_pad = pl.pallas_call(
        _tail_body,
        grid=(N // BN,),
        in_specs=[
            pl.BlockSpec((BN, D), lambda i: (i, 0)),
            pl.BlockSpec((NC, BN, D), lambda i: (0, i, 0)),
            full((D, D)),
            full((1, D)),
            full((1, D)),
            full((D, 128)),
            full((1, 128)),
            full((128, 384)),
            full((1, 384)),
        ],
        out_specs=pl.BlockSpec((BN, 384), lambda i: (i, 0)),
        out_shape=jax.ShapeDtypeStruct((N, 384), jnp.float32),
    )(x, parts, Wo, lng2, lnb2, W_h1p, b_h1p, W_h2p, b_h2p)

    return occ_pad[:, :300]


# speedup vs baseline: 1.0456x; 1.0456x over previous
import functools

import numpy as np
import jax
import jax.numpy as jnp
from jax import lax
from jax.experimental import pallas as pl
from jax.experimental.pallas import tpu as pltpu
from jax.experimental.pallas import tpu_sc as plsc

N = 10000
E = 320000
D = 128
F = 64

NC, NS = 2, 16
NW = NC * NS
GB = 128
EP = 327680
EPWP = EP // NW
NBP = EPWP // GB
CB = 8
NCH = NBP // CB
ACC_ROWS = N + 128

ROWS_PER_TILE = 624

BE = 2560
BN = 2000


def _xm_body(x_ref, wm_ref, bm_ref, o_ref):
    o_ref[:, :] = (
        jnp.dot(x_ref[:, :], wm_ref[:, :], preferred_element_type=jnp.float32)
        + bm_ref[:, :]
    )


_CPOLY = (0.9999710833, -19.73279612, 64.7143697, -82.70120228,
          46.31009229)
_SPOLY = (6.283168274, -41.33792977, 81.47313282, -75.09327405,
          33.95650071)
_NPC = len(_CPOLY)
_LC = np.stack(
    [np.concatenate([np.full(F, c, np.float32), np.full(F, s, np.float32)])
     for c, s in zip(_CPOLY, _SPOLY)]
    + [np.concatenate([np.ones(F, np.float32), np.zeros(F, np.float32)]),
       np.concatenate([np.zeros(F, np.float32), np.ones(F, np.float32)]),
       np.zeros(2 * F, np.float32)])


def _fourier_body(rt_ref, fr_ref, lc_ref, wcs_ref, wlast_ref, bfe_ref,
                  wf2_ref, bf2_ref, o_ref):
    rb = jnp.transpose(rt_ref[:, :], (1, 0))
    h = jnp.zeros((BE, D), dtype=jnp.float32)
    for i in range(3):
        ri = rb[:, i:i + 1]
        t = ri * fr_ref[i:i + 1, :]
        u = t - jnp.round(t)
        v = u * u
        p = lc_ref[_NPC - 1:_NPC, :] + jnp.zeros_like(v)
        for k in range(_NPC - 2, -1, -1):
            p = p * v + lc_ref[k:k + 1, :]
        feat = p * (lc_ref[_NPC:_NPC + 1, :]
                    + lc_ref[_NPC + 1:_NPC + 2, :] * u)
        acc = jnp.dot(feat, wcs_ref[i], preferred_element_type=jnp.float32)
        acc = acc + ri * wlast_ref[i:i + 1, :] + bfe_ref[i:i + 1, :]
        h = h + jnp.maximum(acc, 0.0)
    emb = jnp.dot(h, wf2_ref[:, :], preferred_element_type=jnp.float32)
    o_ref[:, :] = jnp.maximum(emb + bf2_ref[:, :], 0.0)


def _tail_body(x_ref, a_ref, wo_ref, lng_ref, lnb_ref, wh1_ref, bh1_ref,
               wh2_ref, bh2_ref, o_ref):
    agg = a_ref[0] + a_ref[1]
    h = x_ref[:, :] + jnp.dot(agg, wo_ref[:, :],
                              preferred_element_type=jnp.float32)
    mu = jnp.mean(h, axis=1, keepdims=True)
    var = jnp.mean((h - mu) ** 2, axis=1, keepdims=True)
    hn = (h - mu) * lax.rsqrt(var + 1e-5) * lng_ref[:, :] + lnb_ref[:, :]
    t = jnp.maximum(
        jnp.dot(hn, wh1_ref[:, :], preferred_element_type=jnp.float32)
        + bh1_ref[:, :], 0.0)
    o_ref[:, :] = (
        jnp.dot(t, wh2_ref[:, :], preferred_element_type=jnp.float32)
        + bh2_ref[:, :]
    )


def _sc_body(xm_hbm, pre_hbm, src_hbm, dst_hbm, out_hbm,
             src_v, dst_v, gbuf, pbuf, zbuf, acc, gsem, ssem):
    c = lax.axis_index("c")
    s = lax.axis_index("s")
    w = s * NC + c

    zv = jnp.zeros((16,), jnp.float32)
    for i in range(8):
        for g in range(D // 16):
            zbuf[i, pl.ds(g * 16, 16)] = zv

    row0 = s * ROWS_PER_TILE
    nzc = jnp.where(s == NS - 1, (ACC_ROWS - 15 * ROWS_PER_TILE) // 8,
                    ROWS_PER_TILE // 8)

    def zacc(k, carry):
        pltpu.sync_copy(zbuf, acc.at[pl.ds(row0 + k * 8, 8)])
        return carry

    lax.fori_loop(0, nzc, zacc, 0)
    plsc.subcore_barrier()

    def chunk(cc, carry):
        base = (cc * NW + w) * CB
        pltpu.sync_copy(src_hbm.at[pl.ds(base, CB)], src_v)
        pltpu.sync_copy(dst_hbm.at[pl.ds(base, CB)], dst_v)

        g = pltpu.async_copy(xm_hbm.at[src_v.at[0]], gbuf, gsem)
        s_prev = None
        for j in range(CB):
            if s_prev is not None:
                s_prev.wait()
            prow = jnp.minimum(base + j, E // GB - 1) * GB
            pltpu.sync_copy(pre_hbm.at[pl.ds(prow, GB)], pbuf)
            g.wait()

            @plsc.parallel_loop(0, GB, unroll=4)
            def rowfn(i):
                for gg in range(D // 16):
                    sl = pl.ds(gg * 16, 16)
                    pbuf[i, sl] = jnp.maximum(gbuf[i, sl] + pbuf[i, sl], 0.0)

            if j < CB - 1:
                g = pltpu.async_copy(xm_hbm.at[src_v.at[j + 1]], gbuf, gsem)
            s_prev = pltpu.async_copy(pbuf, acc.at[dst_v.at[j]], ssem,
                                      add=True)
        s_prev.wait()
        return carry

    lax.fori_loop(0, NCH, chunk, 0)
    plsc.subcore_barrier()

    @pl.when(s == 0)
    def _flush():
        pltpu.sync_copy(acc.at[pl.ds(0, N)], out_hbm.at[c])


@functools.cache
def _get_sc_call():
    return functools.partial(
        pl.kernel,
    out_type=jax.ShapeDtypeStruct((NC, N, D), jnp.float32),
        mesh=plsc.VectorSubcoreMesh(core_axis_name="c", subcore_axis_name="s",
                                    num_cores=NC, num_subcores=NS),
        scratch_types=[
            pltpu.VMEM((CB, GB), jnp.int32),
            pltpu.VMEM((CB, GB), jnp.int32),
            pltpu.VMEM((GB, D), jnp.float32),
            pltpu.VMEM((GB, D), jnp.float32),
            pltpu.VMEM((8, D), jnp.float32),
            pltpu.VMEM_SHARED((ACC_ROWS, D), jnp.float32),
            pltpu.SemaphoreType.DMA,
            pltpu.SemaphoreType.DMA,
        ],
    )(_sc_body)


def kernel(x, edge_index, r, freqs, W_fe, b_fe, W_f2, b_f2, Wm, bm, Wo,
           ln_g, ln_b, W_h1, b_h1, W_h2, b_h2):
    pad_src = jnp.arange(EP - E, dtype=jnp.int32) % N
    srcp = jnp.concatenate([edge_index[0], pad_src]).reshape(NW * NBP, GB)
    pad_dst = N + (jnp.arange(EP - E, dtype=jnp.int32) % 128)
    dstp = jnp.concatenate([edge_index[1], pad_dst]).reshape(NW * NBP, GB)
    Wcs = W_fe[:, :2 * F, :]
    Wlast = W_fe[:, 2 * F, :]
    bm2 = bm.reshape(1, D)
    bf2 = b_f2.reshape(1, D)
    lng2 = ln_g.reshape(1, D)
    lnb2 = ln_b.reshape(1, D)
    W_h1p = jnp.pad(W_h1, ((0, 0), (0, 28)))
    b_h1p = jnp.pad(b_h1, (0, 28)).reshape(1, 128)
    W_h2p = jnp.pad(W_h2, ((0, 28), (0, 84)))
    b_h2p = jnp.pad(b_h2, (0, 84)).reshape(1, 384)

    full = lambda shape: pl.BlockSpec(shape, lambda i: tuple(0 for _ in shape))

    xm = pl.pallas_call(
        _xm_body,
        grid=(N // BN,),
        in_specs=[
            pl.BlockSpec((BN, D), lambda i: (i, 0)),
            full((D, D)),
            full((1, D)),
        ],
        out_specs=pl.BlockSpec((BN, D), lambda i: (i, 0)),
        out_shape=jax.ShapeDtypeStruct((N, D), jnp.float32),
    )(x, Wm, bm2)

    rt = jnp.swapaxes(r, 0, 1)
    fr2 = jnp.concatenate([freqs, freqs], axis=1)
    pre = pl.pallas_call(
        _fourier_body,
        grid=(E // BE,),
        in_specs=[
            pl.BlockSpec((3, BE), lambda i: (0, i)),
            full((3, 2 * F)),
            full((8, D)),
            full((3, 2 * F, D)),
            full((3, D)),
            full((3, D)),
            full((D, D)),
            full((1, D)),
        ],
        out_specs=pl.BlockSpec((BE, D), lambda i: (i, 0)),
        out_shape=jax.ShapeDtypeStruct((E, D), jnp.float32),
    )(rt, fr2, jnp.asarray(_LC), Wcs, Wlast, b_fe, W_f2, bf2)

    parts = _get_sc_call()(xm, pre, srcp, dstp)

    occ_pad = pl.pallas_call(
        _tail_body,
        grid=(N // BN,),
        in_specs=[
            pl.BlockSpec((BN, D), lambda i: (i, 0)),
            pl.BlockSpec((NC, BN, D), lambda i: (0, i, 0)),
            full((D, D)),
            full((1, D)),
            full((1, D)),
            full((D, 128)),
            full((1, 128)),
            full((128, 384)),
            full((1, 384)),
        ],
        out_specs=pl.BlockSpec((BN, 384), lambda i: (i, 0)),
        out_shape=jax.ShapeDtypeStruct((N, 384), jnp.float32),
    )(x, parts, Wo, lng2, lnb2, W_h1p, b_h1p, W_h2p, b_h2p)

    return occ_pad[:, :300]


# speedup vs baseline: 1.2290x; 1.1754x over previous
import functools

import numpy as np
import jax
import jax.numpy as jnp
from jax import lax
from jax.experimental import pallas as pl
from jax.experimental.pallas import tpu as pltpu
from jax.experimental.pallas import tpu_sc as plsc

N = 10000
E = 320000
D = 128
F = 64

NC, NS = 2, 16
NW = NC * NS
GB = 128
EP = 327680
EPWP = EP // NW
NBP = EPWP // GB
CB = 8
NCH = NBP // CB
ACC_ROWS = N + 128

ROWS_PER_TILE = 624

BE = 2560
BN = 2000


def _xm_body(x_ref, wm_ref, bm_ref, o_ref):
    o_ref[:, :] = (
        jnp.dot(x_ref[:, :], wm_ref[:, :], preferred_element_type=jnp.float32)
        + bm_ref[:, :]
    )


_CPOLY = (0.9999710833, -19.73279612, 64.7143697, -82.70120228,
          46.31009229)
_SPOLY = (6.283168274, -41.33792977, 81.47313282, -75.09327405,
          33.95650071)
_NPC = len(_CPOLY)
_LC = np.stack(
    [np.concatenate([np.full(F, c, np.float32), np.full(F, s, np.float32)])
     for c, s in zip(_CPOLY, _SPOLY)]
    + [np.concatenate([np.ones(F, np.float32), np.zeros(F, np.float32)]),
       np.concatenate([np.zeros(F, np.float32), np.ones(F, np.float32)]),
       np.zeros(2 * F, np.float32)])


def _fourier_body(rt_ref, fr_ref, lc_ref, wcs_ref, wlast_ref, bfe_ref,
                  wf2_ref, bf2_ref, o_ref):
    rb = jnp.transpose(rt_ref[:, :], (1, 0))
    h = jnp.zeros((BE, D), dtype=jnp.float32)
    for i in range(3):
        ri = rb[:, i:i + 1]
        t = ri * fr_ref[i:i + 1, :]
        u = t - jnp.round(t)
        v = u * u
        p = lc_ref[_NPC - 1:_NPC, :] + jnp.zeros_like(v)
        for k in range(_NPC - 2, -1, -1):
            p = p * v + lc_ref[k:k + 1, :]
        feat = p * (lc_ref[_NPC:_NPC + 1, :]
                    + lc_ref[_NPC + 1:_NPC + 2, :] * u)
        acc = jnp.dot(feat, wcs_ref[i], preferred_element_type=jnp.float32)
        acc = acc + ri * wlast_ref[i:i + 1, :] + bfe_ref[i:i + 1, :]
        h = h + jnp.maximum(acc, 0.0)
    emb = jnp.dot(h, wf2_ref[:, :], preferred_element_type=jnp.float32)
    o_ref[:, :] = jnp.maximum(emb + bf2_ref[:, :], 0.0)


def _tail_body(x_ref, a_ref, b_ref, wo_ref, lng_ref, lnb_ref, wh1_ref,
               bh1_ref, wh2_ref, bh2_ref, o_ref):
    agg = (a_ref[0] + a_ref[1]) + (b_ref[0] + b_ref[1])
    h = x_ref[:, :] + jnp.dot(agg, wo_ref[:, :],
                              preferred_element_type=jnp.float32)
    mu = jnp.mean(h, axis=1, keepdims=True)
    var = jnp.mean((h - mu) ** 2, axis=1, keepdims=True)
    hn = (h - mu) * lax.rsqrt(var + 1e-5) * lng_ref[:, :] + lnb_ref[:, :]
    t = jnp.maximum(
        jnp.dot(hn, wh1_ref[:, :], preferred_element_type=jnp.float32)
        + bh1_ref[:, :], 0.0)
    o_ref[:, :] = (
        jnp.dot(t, wh2_ref[:, :], preferred_element_type=jnp.float32)
        + bh2_ref[:, :]
    )


def _make_sc_body(nch, nreal):
  def _sc_body(xm_hbm, pre_hbm, src_hbm, dst_hbm, out_hbm,
               src_v, dst_v, gbuf, pbuf, zbuf, acc, gsem, ssem):
    c = lax.axis_index("c")
    s = lax.axis_index("s")
    w = s * NC + c

    zv = jnp.zeros((16,), jnp.float32)
    for i in range(8):
        for g in range(D // 16):
            zbuf[i, pl.ds(g * 16, 16)] = zv

    row0 = s * ROWS_PER_TILE
    nzc = jnp.where(s == NS - 1, (ACC_ROWS - 15 * ROWS_PER_TILE) // 8,
                    ROWS_PER_TILE // 8)

    def zacc(k, carry):
        pltpu.sync_copy(zbuf, acc.at[pl.ds(row0 + k * 8, 8)])
        return carry

    lax.fori_loop(0, nzc, zacc, 0)
    plsc.subcore_barrier()

    def chunk(cc, carry):
        base = (cc * NW + w) * CB
        pltpu.sync_copy(src_hbm.at[pl.ds(base, CB)], src_v)
        pltpu.sync_copy(dst_hbm.at[pl.ds(base, CB)], dst_v)

        g = pltpu.async_copy(xm_hbm.at[src_v.at[0]], gbuf, gsem)
        s_prev = None
        for j in range(CB):
            if s_prev is not None:
                s_prev.wait()
            prow = jnp.minimum(base + j, nreal - 1) * GB
            pltpu.sync_copy(pre_hbm.at[pl.ds(prow, GB)], pbuf)
            g.wait()

            @plsc.parallel_loop(0, GB, unroll=4)
            def rowfn(i):
                for gg in range(D // 16):
                    sl = pl.ds(gg * 16, 16)
                    pbuf[i, sl] = jnp.maximum(gbuf[i, sl] + pbuf[i, sl], 0.0)

            if j < CB - 1:
                g = pltpu.async_copy(xm_hbm.at[src_v.at[j + 1]], gbuf, gsem)
            s_prev = pltpu.async_copy(pbuf, acc.at[dst_v.at[j]], ssem,
                                      add=True)
        s_prev.wait()
        return carry

    lax.fori_loop(0, nch, chunk, 0)
    plsc.subcore_barrier()

    @pl.when(s == 0)
    def _flush():
        pltpu.sync_copy(acc.at[pl.ds(0, N)], out_hbm.at[c])

  return _sc_body


@functools.cache
def _get_sc_call(nch, nreal):
    return functools.partial(
        pl.kernel,
        out_type=jax.ShapeDtypeStruct((NC, N, D), jnp.float32),
        mesh=plsc.VectorSubcoreMesh(core_axis_name="c", subcore_axis_name="s",
                                    num_cores=NC, num_subcores=NS),
        scratch_types=[
            pltpu.VMEM((CB, GB), jnp.int32),
            pltpu.VMEM((CB, GB), jnp.int32),
            pltpu.VMEM((GB, D), jnp.float32),
            pltpu.VMEM((GB, D), jnp.float32),
            pltpu.VMEM((8, D), jnp.float32),
            pltpu.VMEM_SHARED((ACC_ROWS, D), jnp.float32),
            pltpu.SemaphoreType.DMA,
            pltpu.SemaphoreType.DMA,
        ],
    )(_make_sc_body(nch, nreal))


def kernel(x, edge_index, r, freqs, W_fe, b_fe, W_f2, b_f2, Wm, bm, Wo,
           ln_g, ln_b, W_h1, b_h1, W_h2, b_h2):
    pad_src = jnp.arange(EP - E, dtype=jnp.int32) % N
    srcp = jnp.concatenate([edge_index[0], pad_src]).reshape(NW * NBP, GB)
    pad_dst = N + (jnp.arange(EP - E, dtype=jnp.int32) % 128)
    dstp = jnp.concatenate([edge_index[1], pad_dst]).reshape(NW * NBP, GB)
    Wcs = W_fe[:, :2 * F, :]
    Wlast = W_fe[:, 2 * F, :]
    bm2 = bm.reshape(1, D)
    bf2 = b_f2.reshape(1, D)
    lng2 = ln_g.reshape(1, D)
    lnb2 = ln_b.reshape(1, D)
    W_h1p = jnp.pad(W_h1, ((0, 0), (0, 28)))
    b_h1p = jnp.pad(b_h1, (0, 28)).reshape(1, 128)
    W_h2p = jnp.pad(W_h2, ((0, 28), (0, 84)))
    b_h2p = jnp.pad(b_h2, (0, 84)).reshape(1, 384)

    full = lambda shape: pl.BlockSpec(shape, lambda i: tuple(0 for _ in shape))

    xm = pl.pallas_call(
        _xm_body,
        grid=(N // BN,),
        in_specs=[
            pl.BlockSpec((BN, D), lambda i: (i, 0)),
            full((D, D)),
            full((1, D)),
        ],
        out_specs=pl.BlockSpec((BN, D), lambda i: (i, 0)),
        out_shape=jax.ShapeDtypeStruct((N, D), jnp.float32),
    )(x, Wm, bm2)

    rt = jnp.swapaxes(r, 0, 1)
    fr2 = jnp.concatenate([freqs, freqs], axis=1)

    def fourier(rt_h, ne):
        return pl.pallas_call(
            _fourier_body,
            grid=(ne // BE,),
            in_specs=[
                pl.BlockSpec((3, BE), lambda i: (0, i)),
                full((3, 2 * F)),
                full((8, D)),
                full((3, 2 * F, D)),
                full((3, D)),
                full((3, D)),
                full((D, D)),
                full((1, D)),
            ],
            out_specs=pl.BlockSpec((BE, D), lambda i: (i, 0)),
            out_shape=jax.ShapeDtypeStruct((ne, D), jnp.float32),
        )(rt_h, fr2, jnp.asarray(_LC), Wcs, Wlast, b_fe, W_f2, bf2)

    E0 = 64 * BE
    B_TOT = EP // GB
    B0 = E0 // GB
    pre0 = fourier(rt[:, :E0], E0)
    pre1 = fourier(rt[:, E0:], E - E0)
    parts0 = _get_sc_call(B0 // (NW * CB), B0)(
        xm, pre0, srcp[:B0], dstp[:B0])
    parts1 = _get_sc_call((B_TOT - B0) // (NW * CB), (E - E0) // GB)(
        xm, pre1, srcp[B0:], dstp[B0:])

    occ_pad = pl.pallas_call(
        _tail_body,
        grid=(N // BN,),
        in_specs=[
            pl.BlockSpec((BN, D), lambda i: (i, 0)),
            pl.BlockSpec((NC, BN, D), lambda i: (0, i, 0)),
            pl.BlockSpec((NC, BN, D), lambda i: (0, i, 0)),
            full((D, D)),
            full((1, D)),
            full((1, D)),
            full((D, 128)),
            full((1, 128)),
            full((128, 384)),
            full((1, 384)),
        ],
        out_specs=pl.BlockSpec((BN, 384), lambda i: (i, 0)),
        out_shape=jax.ShapeDtypeStruct((N, 384), jnp.float32),
    )(x, parts0, parts1, Wo, lng2, lnb2, W_h1p, b_h1p, W_h2p, b_h2p)

    return occ_pad[:, :300]


# speedup vs baseline: 1.3313x; 1.0832x over previous
import functools

import numpy as np
import jax
import jax.numpy as jnp
from jax import lax
from jax.experimental import pallas as pl
from jax.experimental.pallas import tpu as pltpu
from jax.experimental.pallas import tpu_sc as plsc

N = 10000
E = 320000
D = 128
F = 64

NC, NS = 2, 16
NW = NC * NS
GB = 128
EP = 327680
EPWP = EP // NW
NBP = EPWP // GB
CB = 4
NCH = NBP // CB
ACC_ROWS = N + 128

ROWS_PER_TILE = 624

BE = 2560
BN = 2000


def _xm_body(x_ref, wm_ref, bm_ref, o_ref):
    o_ref[:, :] = (
        jnp.dot(x_ref[:, :], wm_ref[:, :], preferred_element_type=jnp.float32)
        + bm_ref[:, :]
    )


_CPOLY = (0.9999710833, -19.73279612, 64.7143697, -82.70120228,
          46.31009229)
_SPOLY = (6.283168274, -41.33792977, 81.47313282, -75.09327405,
          33.95650071)
_NPC = len(_CPOLY)
_LC = np.stack(
    [np.concatenate([np.full(F, c, np.float32), np.full(F, s, np.float32)])
     for c, s in zip(_CPOLY, _SPOLY)]
    + [np.concatenate([np.ones(F, np.float32), np.zeros(F, np.float32)]),
       np.concatenate([np.zeros(F, np.float32), np.ones(F, np.float32)]),
       np.zeros(2 * F, np.float32)])


def _fourier_body(rt_ref, fr_ref, lc_ref, wcs_ref, wlast_ref, bfe_ref,
                  wf2_ref, bf2_ref, o_ref):
    rb = jnp.transpose(rt_ref[:, :], (1, 0))
    h = jnp.zeros((BE, D), dtype=jnp.float32)
    for i in range(3):
        ri = rb[:, i:i + 1]
        t = ri * fr_ref[i:i + 1, :]
        u = t - jnp.round(t)
        v = u * u
        p = lc_ref[_NPC - 1:_NPC, :] + jnp.zeros_like(v)
        for k in range(_NPC - 2, -1, -1):
            p = p * v + lc_ref[k:k + 1, :]
        feat = p * (lc_ref[_NPC:_NPC + 1, :]
                    + lc_ref[_NPC + 1:_NPC + 2, :] * u)
        acc = jnp.dot(feat, wcs_ref[i], preferred_element_type=jnp.float32)
        acc = acc + ri * wlast_ref[i:i + 1, :] + bfe_ref[i:i + 1, :]
        h = h + jnp.maximum(acc, 0.0)
    emb = jnp.dot(h, wf2_ref[:, :], preferred_element_type=jnp.float32)
    o_ref[:, :] = jnp.maximum(emb + bf2_ref[:, :], 0.0)


def _tail_body(x_ref, a_ref, b_ref, c_ref, d_ref, wo_ref, lng_ref, lnb_ref,
               wh1_ref, bh1_ref, wh2_ref, bh2_ref, o_ref):
    agg = ((a_ref[0] + a_ref[1]) + (b_ref[0] + b_ref[1])
           + (c_ref[0] + c_ref[1]) + (d_ref[0] + d_ref[1]))
    h = x_ref[:, :] + jnp.dot(agg, wo_ref[:, :],
                              preferred_element_type=jnp.float32)
    mu = jnp.mean(h, axis=1, keepdims=True)
    var = jnp.mean((h - mu) ** 2, axis=1, keepdims=True)
    hn = (h - mu) * lax.rsqrt(var + 1e-5) * lng_ref[:, :] + lnb_ref[:, :]
    t = jnp.maximum(
        jnp.dot(hn, wh1_ref[:, :], preferred_element_type=jnp.float32)
        + bh1_ref[:, :], 0.0)
    o_ref[:, :] = (
        jnp.dot(t, wh2_ref[:, :], preferred_element_type=jnp.float32)
        + bh2_ref[:, :]
    )


def _make_sc_body(nch, nreal):
  def _sc_body(xm_hbm, pre_hbm, src_hbm, dst_hbm, out_hbm,
               src_v, dst_v, gbuf, pbuf, zbuf, acc, gsem, ssem):
    c = lax.axis_index("c")
    s = lax.axis_index("s")
    w = s * NC + c

    zv = jnp.zeros((16,), jnp.float32)
    for i in range(8):
        for g in range(D // 16):
            zbuf[i, pl.ds(g * 16, 16)] = zv

    row0 = s * ROWS_PER_TILE
    nzc = jnp.where(s == NS - 1, (ACC_ROWS - 15 * ROWS_PER_TILE) // 8,
                    ROWS_PER_TILE // 8)

    def zacc(k, carry):
        pltpu.sync_copy(zbuf, acc.at[pl.ds(row0 + k * 8, 8)])
        return carry

    lax.fori_loop(0, nzc, zacc, 0)
    plsc.subcore_barrier()

    def chunk(cc, carry):
        base = (cc * NW + w) * CB
        pltpu.sync_copy(src_hbm.at[pl.ds(base, CB)], src_v)
        pltpu.sync_copy(dst_hbm.at[pl.ds(base, CB)], dst_v)

        g = pltpu.async_copy(xm_hbm.at[src_v.at[0]], gbuf, gsem)
        s_prev = None
        for j in range(CB):
            if s_prev is not None:
                s_prev.wait()
            prow = jnp.minimum(base + j, nreal - 1) * GB
            pltpu.sync_copy(pre_hbm.at[pl.ds(prow, GB)], pbuf)
            g.wait()

            @plsc.parallel_loop(0, GB, unroll=4)
            def rowfn(i):
                for gg in range(D // 16):
                    sl = pl.ds(gg * 16, 16)
                    pbuf[i, sl] = jnp.maximum(gbuf[i, sl] + pbuf[i, sl], 0.0)

            if j < CB - 1:
                g = pltpu.async_copy(xm_hbm.at[src_v.at[j + 1]], gbuf, gsem)
            s_prev = pltpu.async_copy(pbuf, acc.at[dst_v.at[j]], ssem,
                                      add=True)
        s_prev.wait()
        return carry

    lax.fori_loop(0, nch, chunk, 0)
    plsc.subcore_barrier()

    @pl.when(s == 0)
    def _flush():
        pltpu.sync_copy(acc.at[pl.ds(0, N)], out_hbm.at[c])

  return _sc_body


@functools.cache
def _get_sc_call(nch, nreal):
    return functools.partial(
        pl.kernel,
        out_type=jax.ShapeDtypeStruct((NC, N, D), jnp.float32),
        mesh=plsc.VectorSubcoreMesh(core_axis_name="c", subcore_axis_name="s",
                                    num_cores=NC, num_subcores=NS),
        scratch_types=[
            pltpu.VMEM((CB, GB), jnp.int32),
            pltpu.VMEM((CB, GB), jnp.int32),
            pltpu.VMEM((GB, D), jnp.float32),
            pltpu.VMEM((GB, D), jnp.float32),
            pltpu.VMEM((8, D), jnp.float32),
            pltpu.VMEM_SHARED((ACC_ROWS, D), jnp.float32),
            pltpu.SemaphoreType.DMA,
            pltpu.SemaphoreType.DMA,
        ],
    )(_make_sc_body(nch, nreal))


def kernel(x, edge_index, r, freqs, W_fe, b_fe, W_f2, b_f2, Wm, bm, Wo,
           ln_g, ln_b, W_h1, b_h1, W_h2, b_h2):
    pad_src = jnp.arange(EP - E, dtype=jnp.int32) % N
    srcp = jnp.concatenate([edge_index[0], pad_src]).reshape(NW * NBP, GB)
    pad_dst = N + (jnp.arange(EP - E, dtype=jnp.int32) % 128)
    dstp = jnp.concatenate([edge_index[1], pad_dst]).reshape(NW * NBP, GB)
    Wcs = W_fe[:, :2 * F, :]
    Wlast = W_fe[:, 2 * F, :]
    bm2 = bm.reshape(1, D)
    bf2 = b_f2.reshape(1, D)
    lng2 = ln_g.reshape(1, D)
    lnb2 = ln_b.reshape(1, D)
    W_h1p = jnp.pad(W_h1, ((0, 0), (0, 28)))
    b_h1p = jnp.pad(b_h1, (0, 28)).reshape(1, 128)
    W_h2p = jnp.pad(W_h2, ((0, 28), (0, 84)))
    b_h2p = jnp.pad(b_h2, (0, 84)).reshape(1, 384)

    full = lambda shape: pl.BlockSpec(shape, lambda i: tuple(0 for _ in shape))

    xm = pl.pallas_call(
        _xm_body,
        grid=(N // BN,),
        in_specs=[
            pl.BlockSpec((BN, D), lambda i: (i, 0)),
            full((D, D)),
            full((1, D)),
        ],
        out_specs=pl.BlockSpec((BN, D), lambda i: (i, 0)),
        out_shape=jax.ShapeDtypeStruct((N, D), jnp.float32),
    )(x, Wm, bm2)

    rt = jnp.swapaxes(r, 0, 1)
    fr2 = jnp.concatenate([freqs, freqs], axis=1)

    def fourier(rt_h, ne):
        return pl.pallas_call(
            _fourier_body,
            grid=(ne // BE,),
            in_specs=[
                pl.BlockSpec((3, BE), lambda i: (0, i)),
                full((3, 2 * F)),
                full((8, D)),
                full((3, 2 * F, D)),
                full((3, D)),
                full((3, D)),
                full((D, D)),
                full((1, D)),
            ],
            out_specs=pl.BlockSpec((BE, D), lambda i: (i, 0)),
            out_shape=jax.ShapeDtypeStruct((ne, D), jnp.float32),
        )(rt_h, fr2, jnp.asarray(_LC), Wcs, Wlast, b_fe, W_f2, bf2)

    B_TOT = EP // GB
    BQ = B_TOT // 4
    parts_list = []
    for q in range(4):
        e_lo = q * BQ * GB
        e_hi = min((q + 1) * BQ * GB, E)
        pre_q = fourier(rt[:, e_lo:e_hi], e_hi - e_lo)
        parts_list.append(_get_sc_call(BQ // (NW * CB), (e_hi - e_lo) // GB)(
            xm, pre_q, srcp[q * BQ:(q + 1) * BQ], dstp[q * BQ:(q + 1) * BQ]))
    parts0, parts1, parts2, parts3 = parts_list

    occ_pad = pl.pallas_call(
        _tail_body,
        grid=(N // BN,),
        in_specs=[
            pl.BlockSpec((BN, D), lambda i: (i, 0)),
            pl.BlockSpec((NC, BN, D), lambda i: (0, i, 0)),
            pl.BlockSpec((NC, BN, D), lambda i: (0, i, 0)),
            pl.BlockSpec((NC, BN, D), lambda i: (0, i, 0)),
            pl.BlockSpec((NC, BN, D), lambda i: (0, i, 0)),
            full((D, D)),
            full((1, D)),
            full((1, D)),
            full((D, 128)),
            full((1, 128)),
            full((128, 384)),
            full((1, 384)),
        ],
        out_specs=pl.BlockSpec((BN, 384), lambda i: (i, 0)),
        out_shape=jax.ShapeDtypeStruct((N, 384), jnp.float32),
    )(x, parts0, parts1, parts2, parts3, Wo, lng2, lnb2, W_h1p, b_h1p,
      W_h2p, b_h2p)

    return occ_pad[:, :300]
